# Initial kernel scaffold; baseline (speedup 1.0000x reference)
import jax
import jax.numpy as jnp
from jax.experimental import pallas as pl


def kernel(x, edge_index, edge_weight, W1, b1, W2, b2):
    raise NotImplementedError("write your pallas kernel here")


# speedup vs baseline: 8.2867x; 8.2867x over previous
import functools

import jax
import jax.numpy as jnp
from jax import lax
from jax.experimental import pallas as pl
from jax.experimental.pallas import tpu as pltpu
from jax.experimental.pallas import tpu_sc as plsc

NN = 10000
NP = 10240
EE = 320000
DD = 128
CH = 128
NCHUNK = EE // CH
NC, NS = 2, 16
NT = NC * NS
ROWS_PER_TILE = NP // NS
DIS_PER_TILE = NP // NT

_MESH = plsc.VectorSubcoreMesh(
    core_axis_name="c", subcore_axis_name="s", num_cores=NC, num_subcores=NS)


def _rsqrt_newton(d):
    y = lax.bitcast_convert_type(
        jnp.int32(0x5F3759DF)
        - lax.shift_right_logical(lax.bitcast_convert_type(d, jnp.int32), 1),
        jnp.float32)
    for _ in range(4):
        y = y * (1.5 - 0.5 * d * y * y)
    return y


def _prep_body(row_hbm, w_hbm, z1_hbm, disx_hbm,
               row_idx, w_v, deg_v, exp_v, deg_sp):
    c = lax.axis_index("c")
    s = lax.axis_index("s")
    pltpu.sync_copy(z1_hbm, deg_sp.at[pl.ds(s * ROWS_PER_TILE, ROWS_PER_TILE)])
    plsc.subcore_barrier()

    def deg_chunk(i, carry):
        off = (i * NS + s) * CH
        pltpu.sync_copy(row_hbm.at[pl.ds(off, CH)], row_idx)
        pltpu.sync_copy(w_hbm.at[pl.ds(off, CH)], w_v)
        pltpu.sync_copy(w_v, deg_sp.at[row_idx], add=True)
        return carry

    nch = NCHUNK // NS + jnp.where(s < NCHUNK % NS, 1, 0)
    lax.fori_loop(0, nch, deg_chunk, 0)
    plsc.subcore_barrier()

    base = (c * NS + s) * DIS_PER_TILE
    pltpu.sync_copy(deg_sp.at[pl.ds(base, DIS_PER_TILE)], deg_v)

    def newton(t, carry):
        sl = pl.ds(t * 16, 16)
        deg_v[sl] = _rsqrt_newton(deg_v[sl])
        return carry

    lax.fori_loop(0, DIS_PER_TILE // 16, newton, 0)

    def expand(n, carry):
        v = plsc.load_gather(deg_v, [jnp.full((16,), n, jnp.int32)])
        for k in range(DD // 16):
            exp_v[n, pl.ds(k * 16, 16)] = v
        return carry

    lax.fori_loop(0, DIS_PER_TILE, expand, 0)
    pltpu.sync_copy(exp_v, disx_hbm.at[pl.ds(base, DIS_PER_TILE)])


def _spmm_body(tab_hbm, row_hbm, col_hbm, w_hbm, z2_hbm, acc_hbm,
               row_idx, col_idx, w_v, gath_v, sem, acc_sp):
    c = lax.axis_index("c")
    s = lax.axis_index("s")
    wid = c * NS + s
    pltpu.sync_copy(z2_hbm, acc_sp.at[pl.ds(s * ROWS_PER_TILE, ROWS_PER_TILE)])
    plsc.subcore_barrier()

    def chunk(i, carry):
        off = (i * NT + wid) * CH
        pltpu.sync_copy(row_hbm.at[pl.ds(off, CH)], row_idx)
        pltpu.sync_copy(col_hbm.at[pl.ds(off, CH)], col_idx)
        pltpu.sync_copy(w_hbm.at[pl.ds(off, CH)], w_v)
        pltpu.async_copy(tab_hbm.at[row_idx], gath_v, sem).wait()

        def scale(j, carry2):
            wj = plsc.load_gather(w_v, [jnp.full((16,), j, jnp.int32)])
            for k in range(DD // 16):
                sl = pl.ds(k * 16, 16)
                gath_v[j, sl] = gath_v[j, sl] * wj
            return carry2

        lax.fori_loop(0, CH, scale, 0)
        pltpu.sync_copy(gath_v, acc_sp.at[col_idx], add=True)
        return carry

    nch = NCHUNK // NT + jnp.where(wid < NCHUNK % NT, 1, 0)
    lax.fori_loop(0, nch, chunk, 0)
    plsc.subcore_barrier()

    sl = pl.ds(s * ROWS_PER_TILE, ROWS_PER_TILE)
    pltpu.sync_copy(acc_sp.at[sl], acc_hbm.at[c].at[sl])


_SC_PARAMS = pltpu.CompilerParams(needs_layout_passes=False)

_prep = pl.kernel(
    _prep_body,
    out_type=jax.ShapeDtypeStruct((NP, DD), jnp.float32),
    mesh=_MESH,
    compiler_params=_SC_PARAMS,
    scratch_types=[
        pltpu.VMEM((CH,), jnp.int32),
        pltpu.VMEM((CH,), jnp.float32),
        pltpu.VMEM((DIS_PER_TILE,), jnp.float32),
        pltpu.VMEM((DIS_PER_TILE, DD), jnp.float32),
        pltpu.VMEM_SHARED((NP,), jnp.float32),
    ],
)

_spmm = pl.kernel(
    _spmm_body,
    out_type=jax.ShapeDtypeStruct((NC, NP, DD), jnp.float32),
    mesh=_MESH,
    compiler_params=_SC_PARAMS,
    scratch_types=[
        pltpu.VMEM((CH,), jnp.int32),
        pltpu.VMEM((CH,), jnp.int32),
        pltpu.VMEM((CH,), jnp.float32),
        pltpu.VMEM((CH, DD), jnp.float32),
        pltpu.SemaphoreType.DMA,
        pltpu.VMEM_SHARED((NP, DD), jnp.float32),
    ],
)


def _mm1_k(x_ref, w_ref, b_ref, dx_ref, o_ref):
    y = lax.dot_general(x_ref[...], w_ref[...], (((1,), (1,)), ((), ())),
                        preferred_element_type=jnp.float32)
    o_ref[...] = (y + b_ref[...]) * dx_ref[...]


def _mm2_k(p0_ref, p1_ref, dx_ref, w_ref, b_ref, o_ref):
    h = jnp.maximum((p0_ref[...] + p1_ref[...]) * dx_ref[...], 0.0)
    y = lax.dot_general(h, w_ref[...], (((1,), (1,)), ((), ())),
                        preferred_element_type=jnp.float32)
    o_ref[...] = (y + b_ref[...]) * dx_ref[...]


def _fin_k(p0_ref, p1_ref, dx_ref, o_ref):
    o_ref[...] = (p0_ref[...] + p1_ref[...]) * dx_ref[...]


_BLK = 1024
_G = NP // _BLK

_row_spec = pl.BlockSpec((_BLK, DD), lambda i: (i, 0))
_full_spec = pl.BlockSpec((DD, DD), lambda i: (0, 0))
_b_spec = pl.BlockSpec((1, DD), lambda i: (0, 0))

_mm1 = pl.pallas_call(
    _mm1_k, grid=(_G,),
    in_specs=[_row_spec, _full_spec, _b_spec, _row_spec],
    out_specs=_row_spec,
    out_shape=jax.ShapeDtypeStruct((NP, DD), jnp.float32),
)

_mm2 = pl.pallas_call(
    _mm2_k, grid=(_G,),
    in_specs=[_row_spec, _row_spec, _row_spec, _full_spec, _b_spec],
    out_specs=_row_spec,
    out_shape=jax.ShapeDtypeStruct((NP, DD), jnp.float32),
)

_fin = pl.pallas_call(
    _fin_k, grid=(_G,),
    in_specs=[_row_spec, _row_spec, _row_spec],
    out_specs=_row_spec,
    out_shape=jax.ShapeDtypeStruct((NP, DD), jnp.float32),
)


@jax.jit
def kernel(x, edge_index, edge_weight, W1, b1, W2, b2):
    row = edge_index[0].astype(jnp.int32)
    col = edge_index[1].astype(jnp.int32)
    w = edge_weight.astype(jnp.float32)
    xp = jnp.zeros((NP, DD), jnp.float32).at[:NN].set(x)
    z1 = jnp.zeros((ROWS_PER_TILE,), jnp.float32)
    z2 = jnp.zeros((ROWS_PER_TILE, DD), jnp.float32)

    disx = _prep(row, w, z1)
    y1s = _mm1(xp, W1, b1.reshape(1, DD), disx)
    acc1 = _spmm(y1s, row, col, w, z2)
    y2s = _mm2(acc1[0], acc1[1], disx, W2, b2.reshape(1, DD))
    acc2 = _spmm(y2s, row, col, w, z2)
    out = _fin(acc2[0], acc2[1], disx)
    return out[:NN]


# speedup vs baseline: 17.8504x; 2.1541x over previous
import jax
import jax.numpy as jnp
from jax import lax
from jax.experimental import pallas as pl
from jax.experimental.pallas import tpu as pltpu
from jax.experimental.pallas import tpu_sc as plsc

NN = 10000
NP = 10240
EE = 320000
DD = 128
D2 = DD // 2
CH = 128
NCHUNK = EE // CH
NC, NS = 2, 16
ROWS_PER_TILE = NP // NS
DIS_PER_TILE = NP // (NC * NS)
NB = 6
NMAIN = (NCHUNK // NS) // NB * NB
NLEFT = NCHUNK - NS * NMAIN

_MESH = plsc.VectorSubcoreMesh(
    core_axis_name="c", subcore_axis_name="s", num_cores=NC, num_subcores=NS)
_SC_PARAMS = pltpu.CompilerParams(needs_layout_passes=False, use_tc_tiling_on_sc=False)


def _rsqrt_newton(d):
    y = lax.bitcast_convert_type(
        jnp.int32(0x5F3759DF)
        - lax.shift_right_logical(lax.bitcast_convert_type(d, jnp.int32), 1),
        jnp.float32)
    for _ in range(4):
        y = y * (1.5 - 0.5 * d * y * y)
    return y


def _prep_body(pi_hbm, pw_hbm, z1_hbm, disx_hbm, *refs):
    pi = refs[0:NB]
    pw = refs[NB:2 * NB]
    semL = refs[2 * NB:3 * NB]
    semS = refs[3 * NB:4 * NB]
    deg_v, exp_v, deg_sp = refs[4 * NB:4 * NB + 3]
    c = lax.axis_index("c")
    s = lax.axis_index("s")
    pltpu.sync_copy(z1_hbm, deg_sp.at[pl.ds(s * ROWS_PER_TILE, ROWS_PER_TILE)])
    plsc.subcore_barrier()

    def issue_load(i, b):
        g = i * NS + s
        pltpu.async_copy(pi_hbm.at[g], pi[b], semL[b])
        pltpu.async_copy(pw_hbm.at[g], pw[b], semL[b])

    def wait_load(b):
        pltpu.make_async_copy(pi_hbm.at[0], pi[b], semL[b]).wait()
        pltpu.make_async_copy(pw_hbm.at[0], pw[b], semL[b]).wait()

    def wait_scat(b):
        pltpu.make_async_copy(pw_hbm.at[0], pw[b], semS[b]).wait()

    issue_load(0, 0)
    issue_load(1, 1)

    def outer(t, carry):
        for b in range(NB):
            i = t * NB + b
            b2 = (b + 2) % NB

            @pl.when(i < NMAIN - 2)
            def _():
                @pl.when(i >= NB - 2)
                def _():
                    wait_scat(b2)
                issue_load(i + 2, b2)

            wait_load(b)
            pltpu.async_copy(pw[b], deg_sp.at[pi[b].at[0]], semS[b], add=True)
        return carry

    lax.fori_loop(0, NMAIN // NB, outer, 0)
    for b in range(NB):
        wait_scat(b)

    @pl.when(s < NLEFT)
    def _():
        g = NS * NMAIN + s
        pltpu.sync_copy(pi_hbm.at[g], pi[0])
        pltpu.sync_copy(pw_hbm.at[g], pw[0])
        pltpu.sync_copy(pw[0], deg_sp.at[pi[0].at[0]], add=True)

    plsc.subcore_barrier()

    base = (c * NS + s) * DIS_PER_TILE
    pltpu.sync_copy(deg_sp.at[pl.ds(base, DIS_PER_TILE)], deg_v)

    def newton(t, carry):
        sl = pl.ds(t * 16, 16)
        deg_v[sl] = _rsqrt_newton(deg_v[sl])
        return carry

    lax.fori_loop(0, DIS_PER_TILE // 16, newton, 0)

    def expand(n, carry):
        v = plsc.load_gather(deg_v, [jnp.full((16,), n, jnp.int32)])
        for k in range(DD // 16):
            exp_v[n, pl.ds(k * 16, 16)] = v
        return carry

    lax.fori_loop(0, DIS_PER_TILE, expand, 0)
    pltpu.sync_copy(exp_v, disx_hbm.at[pl.ds(base, DIS_PER_TILE)])


def _spmm_body(tab_hbm, pi_hbm, pw_hbm, z2_hbm, acc_hbm, *refs):
    pi = refs[0:NB]
    pw = refs[NB:2 * NB]
    gath = refs[2 * NB:3 * NB]
    semL = refs[3 * NB:4 * NB]
    semG = refs[4 * NB:5 * NB]
    semS = refs[5 * NB:6 * NB]
    acc_sp = refs[6 * NB]
    c = lax.axis_index("c")
    s = lax.axis_index("s")
    tabc = tab_hbm.at[c]
    pltpu.sync_copy(z2_hbm, acc_sp.at[pl.ds(s * ROWS_PER_TILE, ROWS_PER_TILE)])
    plsc.subcore_barrier()

    def issue_load(i, b):
        g = i * NS + s
        pltpu.async_copy(pi_hbm.at[g], pi[b], semL[b])
        pltpu.async_copy(pw_hbm.at[g], pw[b], semL[b])

    def wait_load(b):
        pltpu.make_async_copy(pi_hbm.at[0], pi[b], semL[b]).wait()
        pltpu.make_async_copy(pw_hbm.at[0], pw[b], semL[b]).wait()

    def wait_g(sem, b):
        pltpu.make_async_copy(tabc.at[pl.ds(0, CH)], gath[b], sem).wait()

    def scale(b):
        def body(j, carry):
            for u in range(2):
                jj = j * 2 + u
                wj = plsc.load_gather(pw[b], [jnp.full((16,), jj, jnp.int32)])
                for k in range(D2 // 16):
                    sl = pl.ds(k * 16, 16)
                    gath[b][jj, sl] = gath[b][jj, sl] * wj
            return carry
        lax.fori_loop(0, CH // 2, body, 0)

    issue_load(0, 0)
    issue_load(1, 1)
    wait_load(0)
    pltpu.async_copy(tabc.at[pi[0].at[0]], gath[0], semG[0])

    def outer(t, carry):
        for b in range(NB):
            i = t * NB + b
            b1, b2 = (b + 1) % NB, (b + 2) % NB

            @pl.when(i < NMAIN - 2)
            def _():
                @pl.when(i >= NB - 2)
                def _():
                    wait_g(semS[b2], b2)
                issue_load(i + 2, b2)

            @pl.when(i < NMAIN - 1)
            def _():
                wait_load(b1)
                pltpu.async_copy(tabc.at[pi[b1].at[0]], gath[b1], semG[b1])

            wait_g(semG[b], b)
            scale(b)
            pltpu.async_copy(gath[b], acc_sp.at[pi[b].at[1]], semS[b],
                             add=True)
        return carry

    lax.fori_loop(0, NMAIN // NB, outer, 0)
    for i in range(NMAIN - NB, NMAIN):
        wait_g(semS[i % NB], i % NB)

    @pl.when(s < NLEFT)
    def _():
        g = NS * NMAIN + s
        pltpu.sync_copy(pi_hbm.at[g], pi[0])
        pltpu.sync_copy(pw_hbm.at[g], pw[0])
        pltpu.async_copy(tabc.at[pi[0].at[0]], gath[0], semG[0]).wait()
        scale(0)
        pltpu.sync_copy(gath[0], acc_sp.at[pi[0].at[1]], add=True)

    plsc.subcore_barrier()

    sl = pl.ds(s * ROWS_PER_TILE, ROWS_PER_TILE)
    pltpu.sync_copy(acc_sp.at[sl], acc_hbm.at[c].at[sl])


_prep = pl.kernel(
    _prep_body,
    out_type=jax.ShapeDtypeStruct((NP, DD), jnp.float32),
    mesh=_MESH,
    compiler_params=_SC_PARAMS,
    scratch_types=(
        [pltpu.VMEM((2, CH), jnp.int32) for _ in range(NB)]
        + [pltpu.VMEM((CH,), jnp.float32) for _ in range(NB)]
        + [pltpu.SemaphoreType.DMA for _ in range(2 * NB)]
        + [pltpu.VMEM((DIS_PER_TILE,), jnp.float32),
           pltpu.VMEM((DIS_PER_TILE, DD), jnp.float32),
           pltpu.VMEM_SHARED((NP,), jnp.float32)]
    ),
)

_spmm = pl.kernel(
    _spmm_body,
    out_type=jax.ShapeDtypeStruct((NC, NP, D2), jnp.float32),
    mesh=_MESH,
    compiler_params=_SC_PARAMS,
    scratch_types=(
        [pltpu.VMEM((2, CH), jnp.int32) for _ in range(NB)]
        + [pltpu.VMEM((CH,), jnp.float32) for _ in range(NB)]
        + [pltpu.VMEM((CH, D2), jnp.float32) for _ in range(NB)]
        + [pltpu.SemaphoreType.DMA for _ in range(3 * NB)]
        + [pltpu.VMEM_SHARED((NP, D2), jnp.float32)]
    ),
)


def _mm1_k(x_ref, w_ref, b_ref, dx_ref, o_ref):
    y = lax.dot_general(x_ref[...], w_ref[...], (((1,), (1,)), ((), ())),
                        preferred_element_type=jnp.float32)
    y = (y + b_ref[...]) * dx_ref[...]
    o_ref[0] = y[:, :D2]
    o_ref[1] = y[:, D2:]


def _mm2_k(acc_ref, dx_ref, w_ref, b_ref, o_ref):
    a = jnp.concatenate([acc_ref[0], acc_ref[1]], axis=1)
    h = jnp.maximum(a * dx_ref[...], 0.0)
    y = lax.dot_general(h, w_ref[...], (((1,), (1,)), ((), ())),
                        preferred_element_type=jnp.float32)
    y = (y + b_ref[...]) * dx_ref[...]
    o_ref[0] = y[:, :D2]
    o_ref[1] = y[:, D2:]


def _fin_k(acc_ref, dx_ref, o_ref):
    a = jnp.concatenate([acc_ref[0], acc_ref[1]], axis=1)
    o_ref[...] = a * dx_ref[...]


_BLK = 1024
_G = NP // _BLK

_row_spec = pl.BlockSpec((_BLK, DD), lambda i: (i, 0))
_half_spec = pl.BlockSpec((NC, _BLK, D2), lambda i: (0, i, 0))
_full_spec = pl.BlockSpec((DD, DD), lambda i: (0, 0))
_b_spec = pl.BlockSpec((1, DD), lambda i: (0, 0))

_mm1 = pl.pallas_call(
    _mm1_k, grid=(_G,),
    in_specs=[_row_spec, _full_spec, _b_spec, _row_spec],
    out_specs=_half_spec,
    out_shape=jax.ShapeDtypeStruct((NC, NP, D2), jnp.float32),
)

_mm2 = pl.pallas_call(
    _mm2_k, grid=(_G,),
    in_specs=[_half_spec, _row_spec, _full_spec, _b_spec],
    out_specs=_half_spec,
    out_shape=jax.ShapeDtypeStruct((NC, NP, D2), jnp.float32),
)

_fin = pl.pallas_call(
    _fin_k, grid=(_G,),
    in_specs=[_half_spec, _row_spec],
    out_specs=_row_spec,
    out_shape=jax.ShapeDtypeStruct((NP, DD), jnp.float32),
)


@jax.jit
def kernel(x, edge_index, edge_weight, W1, b1, W2, b2):
    ei = edge_index.astype(jnp.int32)
    pi = jnp.stack([ei[0].reshape(NCHUNK, CH),
                    ei[1].reshape(NCHUNK, CH)], axis=1)
    pw = edge_weight.astype(jnp.float32).reshape(NCHUNK, CH)
    xp = jnp.zeros((NP, DD), jnp.float32).at[:NN].set(x)
    z1 = jnp.zeros((ROWS_PER_TILE,), jnp.float32)
    z2 = jnp.zeros((ROWS_PER_TILE, D2), jnp.float32)

    disx = _prep(pi, pw, z1)
    y1s = _mm1(xp, W1, b1.reshape(1, DD), disx)
    acc1 = _spmm(y1s, pi, pw, z2)
    y2s = _mm2(acc1, disx, W2, b2.reshape(1, DD))
    acc2 = _spmm(y2s, pi, pw, z2)
    out = _fin(acc2, disx)
    return out[:NN]


# speedup vs baseline: 18.4947x; 1.0361x over previous
import jax
import jax.numpy as jnp
from jax import lax
from jax.experimental import pallas as pl
from jax.experimental.pallas import tpu as pltpu
from jax.experimental.pallas import tpu_sc as plsc

NN = 10000
NP = 10240
EE = 320000
DD = 128
D2 = DD // 2
CH = 128
NCHUNK = EE // CH
NC, NS = 2, 16
ROWS_PER_TILE = NP // NS
DIS_PER_TILE = NP // (NC * NS)
NB = 6
NMAIN = (NCHUNK // NS) // NB * NB
NLEFT = NCHUNK - NS * NMAIN

_MESH = plsc.VectorSubcoreMesh(
    core_axis_name="c", subcore_axis_name="s", num_cores=NC, num_subcores=NS)
_SC_PARAMS = pltpu.CompilerParams(needs_layout_passes=False, use_tc_tiling_on_sc=False)


def _rsqrt_newton(d):
    y = lax.bitcast_convert_type(
        jnp.int32(0x5F3759DF)
        - lax.shift_right_logical(lax.bitcast_convert_type(d, jnp.int32), 1),
        jnp.float32)
    for _ in range(4):
        y = y * (1.5 - 0.5 * d * y * y)
    return y


def _prep_body(pi_hbm, pw_hbm, z1_hbm, disx_hbm, *refs):
    pi = refs[0:NB]
    pw = refs[NB:2 * NB]
    semL = refs[2 * NB:3 * NB]
    semS = refs[3 * NB:4 * NB]
    deg_v, exp_v, deg_sp = refs[4 * NB:4 * NB + 3]
    c = lax.axis_index("c")
    s = lax.axis_index("s")
    pltpu.sync_copy(z1_hbm, deg_sp.at[pl.ds(s * ROWS_PER_TILE, ROWS_PER_TILE)])
    plsc.subcore_barrier()

    def issue_load(i, b):
        g = i * NS + s
        pltpu.async_copy(pi_hbm.at[g], pi[b], semL[b])
        pltpu.async_copy(pw_hbm.at[g], pw[b], semL[b])

    def wait_load(b):
        pltpu.make_async_copy(pi_hbm.at[0], pi[b], semL[b]).wait()
        pltpu.make_async_copy(pw_hbm.at[0], pw[b], semL[b]).wait()

    def wait_scat(b):
        pltpu.make_async_copy(pw_hbm.at[0], pw[b], semS[b]).wait()

    issue_load(0, 0)
    issue_load(1, 1)

    def outer(t, carry):
        for b in range(NB):
            i = t * NB + b
            b2 = (b + 2) % NB

            @pl.when(i < NMAIN - 2)
            def _():
                @pl.when(i >= NB - 2)
                def _():
                    wait_scat(b2)
                issue_load(i + 2, b2)

            wait_load(b)
            pltpu.async_copy(pw[b], deg_sp.at[pi[b].at[0]], semS[b], add=True)
        return carry

    lax.fori_loop(0, NMAIN // NB, outer, 0)
    for b in range(NB):
        wait_scat(b)

    @pl.when(s < NLEFT)
    def _():
        g = NS * NMAIN + s
        pltpu.sync_copy(pi_hbm.at[g], pi[0])
        pltpu.sync_copy(pw_hbm.at[g], pw[0])
        pltpu.sync_copy(pw[0], deg_sp.at[pi[0].at[0]], add=True)

    plsc.subcore_barrier()

    base = (c * NS + s) * DIS_PER_TILE
    pltpu.sync_copy(deg_sp.at[pl.ds(base, DIS_PER_TILE)], deg_v)

    def newton(t, carry):
        sl = pl.ds(t * 16, 16)
        deg_v[sl] = _rsqrt_newton(deg_v[sl])
        return carry

    lax.fori_loop(0, DIS_PER_TILE // 16, newton, 0)

    def expand(n, carry):
        v = plsc.load_gather(deg_v, [jnp.full((16,), n, jnp.int32)])
        for k in range(DD // 16):
            exp_v[n, pl.ds(k * 16, 16)] = v
        return carry

    lax.fori_loop(0, DIS_PER_TILE, expand, 0)
    pltpu.sync_copy(exp_v, disx_hbm.at[pl.ds(base, DIS_PER_TILE)])


def _spmm_body(tab_hbm, pi_hbm, pw_hbm, z2_hbm, acc_hbm, *refs):
    pi = refs[0:NB]
    pw = refs[NB:2 * NB]
    gath = refs[2 * NB:3 * NB]
    semL = refs[3 * NB:4 * NB]
    semG = refs[4 * NB:5 * NB]
    semS = refs[5 * NB:6 * NB]
    acc_sp = refs[6 * NB]
    c = lax.axis_index("c")
    s = lax.axis_index("s")
    tabc = tab_hbm.at[c]
    pltpu.sync_copy(z2_hbm, acc_sp.at[pl.ds(s * ROWS_PER_TILE, ROWS_PER_TILE)])
    plsc.subcore_barrier()

    def issue_load(i, b):
        g = i * NS + s
        pltpu.async_copy(pi_hbm.at[g], pi[b], semL[b])
        pltpu.async_copy(pw_hbm.at[g], pw[b], semL[b])

    def wait_load(b):
        pltpu.make_async_copy(pi_hbm.at[0], pi[b], semL[b]).wait()
        pltpu.make_async_copy(pw_hbm.at[0], pw[b], semL[b]).wait()

    def wait_g(sem, b):
        pltpu.make_async_copy(tabc.at[pl.ds(0, CH)], gath[b], sem).wait()

    def scale(b):
        def body(j, carry):
            for u in range(4):
                jj = j * 4 + u
                wj = plsc.load_gather(pw[b], [jnp.full((16,), jj, jnp.int32)])
                for k in range(D2 // 16):
                    sl = pl.ds(k * 16, 16)
                    gath[b][jj, sl] = gath[b][jj, sl] * wj
            return carry
        lax.fori_loop(0, CH // 4, body, 0)

    issue_load(0, 0)
    issue_load(1, 1)
    issue_load(2, 2)
    wait_load(0)
    pltpu.async_copy(tabc.at[pi[0].at[0]], gath[0], semG[0])
    wait_load(1)
    pltpu.async_copy(tabc.at[pi[1].at[0]], gath[1], semG[1])

    def outer(t, carry):
        for b in range(NB):
            i = t * NB + b
            b2, b3 = (b + 2) % NB, (b + 3) % NB

            @pl.when(i < NMAIN - 3)
            def _():
                @pl.when(i >= NB - 3)
                def _():
                    wait_g(semS[b3], b3)
                issue_load(i + 3, b3)

            @pl.when(i < NMAIN - 2)
            def _():
                wait_load(b2)
                pltpu.async_copy(tabc.at[pi[b2].at[0]], gath[b2], semG[b2])

            wait_g(semG[b], b)
            scale(b)
            pltpu.async_copy(gath[b], acc_sp.at[pi[b].at[1]], semS[b],
                             add=True)
        return carry

    lax.fori_loop(0, NMAIN // NB, outer, 0)
    for i in range(NMAIN - NB, NMAIN):
        wait_g(semS[i % NB], i % NB)

    @pl.when(s < NLEFT)
    def _():
        g = NS * NMAIN + s
        pltpu.sync_copy(pi_hbm.at[g], pi[0])
        pltpu.sync_copy(pw_hbm.at[g], pw[0])
        pltpu.async_copy(tabc.at[pi[0].at[0]], gath[0], semG[0]).wait()
        scale(0)
        pltpu.sync_copy(gath[0], acc_sp.at[pi[0].at[1]], add=True)

    plsc.subcore_barrier()

    sl = pl.ds(s * ROWS_PER_TILE, ROWS_PER_TILE)
    pltpu.sync_copy(acc_sp.at[sl], acc_hbm.at[c].at[sl])


_prep = pl.kernel(
    _prep_body,
    out_type=jax.ShapeDtypeStruct((NP, DD), jnp.float32),
    mesh=_MESH,
    compiler_params=_SC_PARAMS,
    scratch_types=(
        [pltpu.VMEM((2, CH), jnp.int32) for _ in range(NB)]
        + [pltpu.VMEM((CH,), jnp.float32) for _ in range(NB)]
        + [pltpu.SemaphoreType.DMA for _ in range(2 * NB)]
        + [pltpu.VMEM((DIS_PER_TILE,), jnp.float32),
           pltpu.VMEM((DIS_PER_TILE, DD), jnp.float32),
           pltpu.VMEM_SHARED((NP,), jnp.float32)]
    ),
)

_spmm = pl.kernel(
    _spmm_body,
    out_type=jax.ShapeDtypeStruct((NC, NP, D2), jnp.float32),
    mesh=_MESH,
    compiler_params=_SC_PARAMS,
    scratch_types=(
        [pltpu.VMEM((2, CH), jnp.int32) for _ in range(NB)]
        + [pltpu.VMEM((CH,), jnp.float32) for _ in range(NB)]
        + [pltpu.VMEM((CH, D2), jnp.float32) for _ in range(NB)]
        + [pltpu.SemaphoreType.DMA for _ in range(3 * NB)]
        + [pltpu.VMEM_SHARED((NP, D2), jnp.float32)]
    ),
)


def _mm1_k(x_ref, w_ref, b_ref, dx_ref, o_ref):
    y = lax.dot_general(x_ref[...], w_ref[...], (((1,), (1,)), ((), ())),
                        preferred_element_type=jnp.float32)
    y = (y + b_ref[...]) * dx_ref[...]
    o_ref[0] = y[:, :D2]
    o_ref[1] = y[:, D2:]


def _mm2_k(acc_ref, dx_ref, w_ref, b_ref, o_ref):
    a = jnp.concatenate([acc_ref[0], acc_ref[1]], axis=1)
    h = jnp.maximum(a * dx_ref[...], 0.0)
    y = lax.dot_general(h, w_ref[...], (((1,), (1,)), ((), ())),
                        preferred_element_type=jnp.float32)
    y = (y + b_ref[...]) * dx_ref[...]
    o_ref[0] = y[:, :D2]
    o_ref[1] = y[:, D2:]


def _fin_k(acc_ref, dx_ref, o_ref):
    a = jnp.concatenate([acc_ref[0], acc_ref[1]], axis=1)
    o_ref[...] = a * dx_ref[...]


_BLK = 1024
_G = NP // _BLK

_row_spec = pl.BlockSpec((_BLK, DD), lambda i: (i, 0))
_half_spec = pl.BlockSpec((NC, _BLK, D2), lambda i: (0, i, 0))
_full_spec = pl.BlockSpec((DD, DD), lambda i: (0, 0))
_b_spec = pl.BlockSpec((1, DD), lambda i: (0, 0))

_mm1 = pl.pallas_call(
    _mm1_k, grid=(_G,),
    in_specs=[_row_spec, _full_spec, _b_spec, _row_spec],
    out_specs=_half_spec,
    out_shape=jax.ShapeDtypeStruct((NC, NP, D2), jnp.float32),
)

_mm2 = pl.pallas_call(
    _mm2_k, grid=(_G,),
    in_specs=[_half_spec, _row_spec, _full_spec, _b_spec],
    out_specs=_half_spec,
    out_shape=jax.ShapeDtypeStruct((NC, NP, D2), jnp.float32),
)

_fin = pl.pallas_call(
    _fin_k, grid=(_G,),
    in_specs=[_half_spec, _row_spec],
    out_specs=_row_spec,
    out_shape=jax.ShapeDtypeStruct((NP, DD), jnp.float32),
)


@jax.jit
def kernel(x, edge_index, edge_weight, W1, b1, W2, b2):
    ei = edge_index.astype(jnp.int32)
    pi = jnp.stack([ei[0].reshape(NCHUNK, CH),
                    ei[1].reshape(NCHUNK, CH)], axis=1)
    pw = edge_weight.astype(jnp.float32).reshape(NCHUNK, CH)
    xp = jnp.zeros((NP, DD), jnp.float32).at[:NN].set(x)
    z1 = jnp.zeros((ROWS_PER_TILE,), jnp.float32)
    z2 = jnp.zeros((ROWS_PER_TILE, D2), jnp.float32)

    disx = _prep(pi, pw, z1)
    y1s = _mm1(xp, W1, b1.reshape(1, DD), disx)
    acc1 = _spmm(y1s, pi, pw, z2)
    y2s = _mm2(acc1, disx, W2, b2.reshape(1, DD))
    acc2 = _spmm(y2s, pi, pw, z2)
    out = _fin(acc2, disx)
    return out[:NN]


# speedup vs baseline: 18.9300x; 1.0235x over previous
import jax
import jax.numpy as jnp
from jax import lax
from jax.experimental import pallas as pl
from jax.experimental.pallas import tpu as pltpu
from jax.experimental.pallas import tpu_sc as plsc

NN = 10000
NP = 10240
EE = 320000
DD = 128
D2 = DD // 2
CH = 128
NCHUNK = EE // CH
NC, NS = 2, 16
ROWS_PER_TILE = NP // NS
DIS_PER_TILE = NP // (NC * NS)
NB = 6
NMAIN = (NCHUNK // NS) // NB * NB
NLEFT = NCHUNK - NS * NMAIN

_MESH = plsc.VectorSubcoreMesh(
    core_axis_name="c", subcore_axis_name="s", num_cores=NC, num_subcores=NS)
_SC_PARAMS = pltpu.CompilerParams(needs_layout_passes=False, use_tc_tiling_on_sc=False)


def _rsqrt_newton(d):
    y = lax.bitcast_convert_type(
        jnp.int32(0x5F3759DF)
        - lax.shift_right_logical(lax.bitcast_convert_type(d, jnp.int32), 1),
        jnp.float32)
    for _ in range(4):
        y = y * (1.5 - 0.5 * d * y * y)
    return y


def _prep_body(pi_hbm, pw_hbm, z1_hbm, disx_hbm, *refs):
    pi = refs[0:NB]
    pw = refs[NB:2 * NB]
    semL = refs[2 * NB:3 * NB]
    semS = refs[3 * NB:4 * NB]
    deg_v, exp_v, deg_sp = refs[4 * NB:4 * NB + 3]
    c = lax.axis_index("c")
    s = lax.axis_index("s")
    pltpu.sync_copy(z1_hbm, deg_sp.at[pl.ds(s * ROWS_PER_TILE, ROWS_PER_TILE)])
    plsc.subcore_barrier()

    def issue_load(i, b):
        g = i * NS + s
        pltpu.async_copy(pi_hbm.at[0].at[g], pi[b].at[0], semL[b])
        pltpu.async_copy(pw_hbm.at[g], pw[b], semL[b])

    def wait_load(b):
        pltpu.make_async_copy(pi_hbm.at[0].at[0], pi[b].at[0], semL[b]).wait()
        pltpu.make_async_copy(pw_hbm.at[0], pw[b], semL[b]).wait()

    def wait_scat(b):
        pltpu.make_async_copy(pw_hbm.at[0], pw[b], semS[b]).wait()

    issue_load(0, 0)
    issue_load(1, 1)

    def outer(t, carry):
        for b in range(NB):
            i = t * NB + b
            b2 = (b + 2) % NB

            @pl.when(i < NMAIN - 2)
            def _():
                @pl.when(i >= NB - 2)
                def _():
                    wait_scat(b2)
                issue_load(i + 2, b2)

            wait_load(b)
            pltpu.async_copy(pw[b], deg_sp.at[pi[b].at[0]], semS[b], add=True)
        return carry

    lax.fori_loop(0, NMAIN // NB, outer, 0)
    for b in range(NB):
        wait_scat(b)

    @pl.when(s < NLEFT)
    def _():
        g = NS * NMAIN + s
        pltpu.sync_copy(pi_hbm.at[0].at[g], pi[0].at[0])
        pltpu.sync_copy(pw_hbm.at[g], pw[0])
        pltpu.sync_copy(pw[0], deg_sp.at[pi[0].at[0]], add=True)

    plsc.subcore_barrier()

    base = (c * NS + s) * DIS_PER_TILE
    pltpu.sync_copy(deg_sp.at[pl.ds(base, DIS_PER_TILE)], deg_v)

    def newton(t, carry):
        sl = pl.ds(t * 16, 16)
        deg_v[sl] = _rsqrt_newton(deg_v[sl])
        return carry

    lax.fori_loop(0, DIS_PER_TILE // 16, newton, 0)

    def expand(n, carry):
        v = plsc.load_gather(deg_v, [jnp.full((16,), n, jnp.int32)])
        for k in range(DD // 16):
            exp_v[n, pl.ds(k * 16, 16)] = v
        return carry

    lax.fori_loop(0, DIS_PER_TILE, expand, 0)
    pltpu.sync_copy(exp_v, disx_hbm.at[pl.ds(base, DIS_PER_TILE)])


def _spmm_body(tab_hbm, pi_hbm, pw_hbm, z2_hbm, acc_hbm, *refs):
    pi = refs[0:NB]
    pw = refs[NB:2 * NB]
    gath = refs[2 * NB:3 * NB]
    semL = refs[3 * NB:4 * NB]
    semG = refs[4 * NB:5 * NB]
    semS = refs[5 * NB:6 * NB]
    acc_sp = refs[6 * NB]
    c = lax.axis_index("c")
    s = lax.axis_index("s")
    tabc = tab_hbm.at[c]
    pltpu.sync_copy(z2_hbm, acc_sp.at[pl.ds(s * ROWS_PER_TILE, ROWS_PER_TILE)])
    plsc.subcore_barrier()

    def issue_load(i, b):
        g = i * NS + s
        pltpu.async_copy(pi_hbm.at[0].at[g], pi[b].at[0], semL[b])
        pltpu.async_copy(pi_hbm.at[1].at[g], pi[b].at[1], semL[b])
        pltpu.async_copy(pw_hbm.at[g], pw[b], semL[b])

    def wait_load(b):
        pltpu.make_async_copy(pi_hbm.at[0].at[0], pi[b].at[0], semL[b]).wait()
        pltpu.make_async_copy(pi_hbm.at[0].at[0], pi[b].at[1], semL[b]).wait()
        pltpu.make_async_copy(pw_hbm.at[0], pw[b], semL[b]).wait()

    def wait_g(sem, b):
        pltpu.make_async_copy(tabc.at[pl.ds(0, CH)], gath[b], sem).wait()

    def scale(b):
        def body(j, carry):
            for u in range(4):
                jj = j * 4 + u
                wj = plsc.load_gather(pw[b], [jnp.full((16,), jj, jnp.int32)])
                for k in range(D2 // 16):
                    sl = pl.ds(k * 16, 16)
                    gath[b][jj, sl] = gath[b][jj, sl] * wj
            return carry
        lax.fori_loop(0, CH // 4, body, 0)

    issue_load(0, 0)
    issue_load(1, 1)
    issue_load(2, 2)
    wait_load(0)
    pltpu.async_copy(tabc.at[pi[0].at[0]], gath[0], semG[0])
    wait_load(1)
    pltpu.async_copy(tabc.at[pi[1].at[0]], gath[1], semG[1])

    def outer(t, carry):
        for b in range(NB):
            i = t * NB + b
            b2, b3 = (b + 2) % NB, (b + 3) % NB

            @pl.when(i < NMAIN - 3)
            def _():
                @pl.when(i >= NB - 3)
                def _():
                    wait_g(semS[b3], b3)
                issue_load(i + 3, b3)

            @pl.when(i < NMAIN - 2)
            def _():
                wait_load(b2)
                pltpu.async_copy(tabc.at[pi[b2].at[0]], gath[b2], semG[b2])

            wait_g(semG[b], b)
            scale(b)
            pltpu.async_copy(gath[b], acc_sp.at[pi[b].at[1]], semS[b],
                             add=True)
        return carry

    lax.fori_loop(0, NMAIN // NB, outer, 0)
    for i in range(NMAIN - NB, NMAIN):
        wait_g(semS[i % NB], i % NB)

    @pl.when(s < NLEFT)
    def _():
        g = NS * NMAIN + s
        pltpu.sync_copy(pi_hbm.at[0].at[g], pi[0].at[0])
        pltpu.sync_copy(pi_hbm.at[1].at[g], pi[0].at[1])
        pltpu.sync_copy(pw_hbm.at[g], pw[0])
        pltpu.async_copy(tabc.at[pi[0].at[0]], gath[0], semG[0]).wait()
        scale(0)
        pltpu.sync_copy(gath[0], acc_sp.at[pi[0].at[1]], add=True)

    plsc.subcore_barrier()

    sl = pl.ds(s * ROWS_PER_TILE, ROWS_PER_TILE)
    pltpu.sync_copy(acc_sp.at[sl], acc_hbm.at[c].at[sl])


_prep = pl.kernel(
    _prep_body,
    out_type=jax.ShapeDtypeStruct((NP, DD), jnp.float32),
    mesh=_MESH,
    compiler_params=_SC_PARAMS,
    scratch_types=(
        [pltpu.VMEM((2, CH), jnp.int32) for _ in range(NB)]
        + [pltpu.VMEM((CH,), jnp.float32) for _ in range(NB)]
        + [pltpu.SemaphoreType.DMA for _ in range(2 * NB)]
        + [pltpu.VMEM((DIS_PER_TILE,), jnp.float32),
           pltpu.VMEM((DIS_PER_TILE, DD), jnp.float32),
           pltpu.VMEM_SHARED((NP,), jnp.float32)]
    ),
)

_spmm = pl.kernel(
    _spmm_body,
    out_type=jax.ShapeDtypeStruct((NC, NP, D2), jnp.float32),
    mesh=_MESH,
    compiler_params=_SC_PARAMS,
    scratch_types=(
        [pltpu.VMEM((2, CH), jnp.int32) for _ in range(NB)]
        + [pltpu.VMEM((CH,), jnp.float32) for _ in range(NB)]
        + [pltpu.VMEM((CH, D2), jnp.float32) for _ in range(NB)]
        + [pltpu.SemaphoreType.DMA for _ in range(3 * NB)]
        + [pltpu.VMEM_SHARED((NP, D2), jnp.float32)]
    ),
)


def _mm1_k(x_ref, w_ref, b_ref, dx_ref, o_ref):
    y = lax.dot_general(x_ref[...], w_ref[...], (((1,), (1,)), ((), ())),
                        preferred_element_type=jnp.float32)
    y = (y + b_ref[...]) * dx_ref[...]
    o_ref[0] = y[:, :D2]
    o_ref[1] = y[:, D2:]


def _mm2_k(acc_ref, dx_ref, w_ref, b_ref, o_ref):
    a = jnp.concatenate([acc_ref[0], acc_ref[1]], axis=1)
    h = jnp.maximum(a * dx_ref[...], 0.0)
    y = lax.dot_general(h, w_ref[...], (((1,), (1,)), ((), ())),
                        preferred_element_type=jnp.float32)
    y = (y + b_ref[...]) * dx_ref[...]
    o_ref[0] = y[:, :D2]
    o_ref[1] = y[:, D2:]


def _fin_k(acc_ref, dx_ref, o_ref):
    a = jnp.concatenate([acc_ref[0], acc_ref[1]], axis=1)
    o_ref[...] = a * dx_ref[...]


_BLK = 1000
_G = NN // _BLK

_row_spec = pl.BlockSpec((_BLK, DD), lambda i: (i, 0))
_half_spec = pl.BlockSpec((NC, _BLK, D2), lambda i: (0, i, 0))
_full_spec = pl.BlockSpec((DD, DD), lambda i: (0, 0))
_b_spec = pl.BlockSpec((1, DD), lambda i: (0, 0))

_mm1 = pl.pallas_call(
    _mm1_k, grid=(_G,),
    in_specs=[_row_spec, _full_spec, _b_spec, _row_spec],
    out_specs=_half_spec,
    out_shape=jax.ShapeDtypeStruct((NC, NN, D2), jnp.float32),
)

_mm2 = pl.pallas_call(
    _mm2_k, grid=(_G,),
    in_specs=[_half_spec, _row_spec, _full_spec, _b_spec],
    out_specs=_half_spec,
    out_shape=jax.ShapeDtypeStruct((NC, NN, D2), jnp.float32),
)

_fin = pl.pallas_call(
    _fin_k, grid=(_G,),
    in_specs=[_half_spec, _row_spec],
    out_specs=_row_spec,
    out_shape=jax.ShapeDtypeStruct((NN, DD), jnp.float32),
)


@jax.jit
def kernel(x, edge_index, edge_weight, W1, b1, W2, b2):
    pi = edge_index.astype(jnp.int32).reshape(2, NCHUNK, CH)
    pw = edge_weight.astype(jnp.float32).reshape(NCHUNK, CH)
    z1 = jnp.zeros((ROWS_PER_TILE,), jnp.float32)
    z2 = jnp.zeros((ROWS_PER_TILE, D2), jnp.float32)

    disx = _prep(pi, pw, z1)
    y1s = _mm1(x, W1, b1.reshape(1, DD), disx)
    acc1 = _spmm(y1s, pi, pw, z2)
    y2s = _mm2(acc1, disx, W2, b2.reshape(1, DD))
    acc2 = _spmm(y2s, pi, pw, z2)
    return _fin(acc2, disx)


# speedup vs baseline: 23.3759x; 1.2349x over previous
import jax
import jax.numpy as jnp
from jax import lax
from jax.experimental import pallas as pl
from jax.experimental.pallas import tpu as pltpu
from jax.experimental.pallas import tpu_sc as plsc

NN = 10000
NP = 10240
EE = 320000
DD = 128
D2 = DD // 2
CH = 128
NCHUNK = EE // CH
NC, NS = 2, 16
ROWS_PER_TILE = NP // NS
DIS_PER_TILE = NP // (NC * NS)
NB = 6
NMAIN = (NCHUNK // NS) // NB * NB
NLEFT = NCHUNK - NS * NMAIN

_MESH = plsc.VectorSubcoreMesh(
    core_axis_name="c", subcore_axis_name="s", num_cores=NC, num_subcores=NS)
_SC_PARAMS = pltpu.CompilerParams(needs_layout_passes=False, use_tc_tiling_on_sc=False)


def _rsqrt_newton(d):
    y = lax.bitcast_convert_type(
        jnp.int32(0x5F3759DF)
        - lax.shift_right_logical(lax.bitcast_convert_type(d, jnp.int32), 1),
        jnp.float32)
    for _ in range(4):
        y = y * (1.5 - 0.5 * d * y * y)
    return y


def _prep_body(pi_hbm, pw_hbm, z1_hbm, disx_hbm, *refs):
    pi = refs[0:NB]
    pw = refs[NB:2 * NB]
    semL = refs[2 * NB:3 * NB]
    semS = refs[3 * NB:4 * NB]
    deg_v, exp_v, deg_sp = refs[4 * NB:4 * NB + 3]
    c = lax.axis_index("c")
    s = lax.axis_index("s")
    pltpu.sync_copy(z1_hbm, deg_sp.at[pl.ds(s * ROWS_PER_TILE, ROWS_PER_TILE)])
    plsc.subcore_barrier()

    def issue_load(i, b):
        g = i * NS + s
        pltpu.async_copy(pi_hbm.at[0].at[g], pi[b].at[0], semL[b])
        pltpu.async_copy(pw_hbm.at[g], pw[b], semL[b])

    def wait_load(b):
        pltpu.make_async_copy(pi_hbm.at[0].at[0], pi[b].at[0], semL[b]).wait()
        pltpu.make_async_copy(pw_hbm.at[0], pw[b], semL[b]).wait()

    def wait_scat(b):
        pltpu.make_async_copy(pw_hbm.at[0], pw[b], semS[b]).wait()

    issue_load(0, 0)
    issue_load(1, 1)

    def outer(t, carry):
        for b in range(NB):
            i = t * NB + b
            b2 = (b + 2) % NB

            @pl.when(i < NMAIN - 2)
            def _():
                @pl.when(i >= NB - 2)
                def _():
                    wait_scat(b2)
                issue_load(i + 2, b2)

            wait_load(b)
            pltpu.async_copy(pw[b], deg_sp.at[pi[b].at[0]], semS[b], add=True)
        return carry

    lax.fori_loop(0, NMAIN // NB, outer, 0)
    for b in range(NB):
        wait_scat(b)

    @pl.when(s < NLEFT)
    def _():
        g = NS * NMAIN + s
        pltpu.sync_copy(pi_hbm.at[0].at[g], pi[0].at[0])
        pltpu.sync_copy(pw_hbm.at[g], pw[0])
        pltpu.sync_copy(pw[0], deg_sp.at[pi[0].at[0]], add=True)

    plsc.subcore_barrier()

    base = (c * NS + s) * DIS_PER_TILE
    pltpu.sync_copy(deg_sp.at[pl.ds(base, DIS_PER_TILE)], deg_v)

    def newton(t, carry):
        sl = pl.ds(t * 16, 16)
        deg_v[sl] = _rsqrt_newton(deg_v[sl])
        return carry

    lax.fori_loop(0, DIS_PER_TILE // 16, newton, 0)

    def expand(n, carry):
        v = plsc.load_gather(deg_v, [jnp.full((16,), n, jnp.int32)])
        for k in range(DD // 16):
            exp_v[n, pl.ds(k * 16, 16)] = v
        return carry

    lax.fori_loop(0, DIS_PER_TILE, expand, 0)
    pltpu.sync_copy(exp_v, disx_hbm.at[pl.ds(base, DIS_PER_TILE)])


def _spmm_body(tab_hbm, pi_hbm, pw_hbm, z2_hbm, acc_hbm, *refs):
    pi = refs[0:NB]
    pw = refs[NB:2 * NB]
    gath = refs[2 * NB:3 * NB]
    semL = refs[3 * NB:4 * NB]
    semG = refs[4 * NB:5 * NB]
    semS = refs[5 * NB:6 * NB]
    acc_sp = refs[6 * NB]
    c = lax.axis_index("c")
    s = lax.axis_index("s")
    tabc = tab_hbm.at[c]
    pltpu.sync_copy(z2_hbm, acc_sp.at[pl.ds(s * ROWS_PER_TILE, ROWS_PER_TILE)])
    plsc.subcore_barrier()

    def issue_load(i, b):
        g = i * NS + s
        pltpu.async_copy(pi_hbm.at[0].at[g], pi[b].at[0], semL[b])
        pltpu.async_copy(pi_hbm.at[1].at[g], pi[b].at[1], semL[b])
        pltpu.async_copy(pw_hbm.at[g], pw[b], semL[b])

    def wait_load(b):
        pltpu.make_async_copy(pi_hbm.at[0].at[0], pi[b].at[0], semL[b]).wait()
        pltpu.make_async_copy(pi_hbm.at[0].at[0], pi[b].at[1], semL[b]).wait()
        pltpu.make_async_copy(pw_hbm.at[0], pw[b], semL[b]).wait()

    def wait_g(sem, b):
        pltpu.make_async_copy(tabc.at[pl.ds(0, CH)], gath[b], sem).wait()

    def scale(b):
        @plsc.parallel_loop(0, CH, step=1, unroll=4)
        def body(jj):
            wj = plsc.load_gather(pw[b], [jnp.full((16,), jj, jnp.int32)])
            for k in range(D2 // 16):
                sl = pl.ds(k * 16, 16)
                gath[b][jj, sl] = gath[b][jj, sl] * wj

    issue_load(0, 0)
    issue_load(1, 1)
    issue_load(2, 2)
    wait_load(0)
    pltpu.async_copy(tabc.at[pi[0].at[0]], gath[0], semG[0])
    wait_load(1)
    pltpu.async_copy(tabc.at[pi[1].at[0]], gath[1], semG[1])

    def outer(t, carry):
        for b in range(NB):
            i = t * NB + b
            b2, b3 = (b + 2) % NB, (b + 3) % NB

            @pl.when(i < NMAIN - 3)
            def _():
                @pl.when(i >= NB - 3)
                def _():
                    wait_g(semS[b3], b3)
                issue_load(i + 3, b3)

            @pl.when(i < NMAIN - 2)
            def _():
                wait_load(b2)
                pltpu.async_copy(tabc.at[pi[b2].at[0]], gath[b2], semG[b2])

            wait_g(semG[b], b)
            scale(b)
            pltpu.async_copy(gath[b], acc_sp.at[pi[b].at[1]], semS[b],
                             add=True)
        return carry

    lax.fori_loop(0, NMAIN // NB, outer, 0)
    for i in range(NMAIN - NB, NMAIN):
        wait_g(semS[i % NB], i % NB)

    @pl.when(s < NLEFT)
    def _():
        g = NS * NMAIN + s
        pltpu.sync_copy(pi_hbm.at[0].at[g], pi[0].at[0])
        pltpu.sync_copy(pi_hbm.at[1].at[g], pi[0].at[1])
        pltpu.sync_copy(pw_hbm.at[g], pw[0])
        pltpu.async_copy(tabc.at[pi[0].at[0]], gath[0], semG[0]).wait()
        scale(0)
        pltpu.sync_copy(gath[0], acc_sp.at[pi[0].at[1]], add=True)

    plsc.subcore_barrier()

    sl = pl.ds(s * ROWS_PER_TILE, ROWS_PER_TILE)
    pltpu.sync_copy(acc_sp.at[sl], acc_hbm.at[c].at[sl])


_prep = pl.kernel(
    _prep_body,
    out_type=jax.ShapeDtypeStruct((NP, DD), jnp.float32),
    mesh=_MESH,
    compiler_params=_SC_PARAMS,
    scratch_types=(
        [pltpu.VMEM((2, CH), jnp.int32) for _ in range(NB)]
        + [pltpu.VMEM((CH,), jnp.float32) for _ in range(NB)]
        + [pltpu.SemaphoreType.DMA for _ in range(2 * NB)]
        + [pltpu.VMEM((DIS_PER_TILE,), jnp.float32),
           pltpu.VMEM((DIS_PER_TILE, DD), jnp.float32),
           pltpu.VMEM_SHARED((NP,), jnp.float32)]
    ),
)

_spmm = pl.kernel(
    _spmm_body,
    out_type=jax.ShapeDtypeStruct((NC, NP, D2), jnp.float32),
    mesh=_MESH,
    compiler_params=_SC_PARAMS,
    scratch_types=(
        [pltpu.VMEM((2, CH), jnp.int32) for _ in range(NB)]
        + [pltpu.VMEM((CH,), jnp.float32) for _ in range(NB)]
        + [pltpu.VMEM((CH, D2), jnp.float32) for _ in range(NB)]
        + [pltpu.SemaphoreType.DMA for _ in range(3 * NB)]
        + [pltpu.VMEM_SHARED((NP, D2), jnp.float32)]
    ),
)


def _mm1_k(x_ref, w_ref, b_ref, dx_ref, o_ref):
    y = lax.dot_general(x_ref[...], w_ref[...], (((1,), (1,)), ((), ())),
                        preferred_element_type=jnp.float32)
    y = (y + b_ref[...]) * dx_ref[...]
    o_ref[0] = y[:, :D2]
    o_ref[1] = y[:, D2:]


def _mm2_k(acc_ref, dx_ref, w_ref, b_ref, o_ref):
    a = jnp.concatenate([acc_ref[0], acc_ref[1]], axis=1)
    h = jnp.maximum(a * dx_ref[...], 0.0)
    y = lax.dot_general(h, w_ref[...], (((1,), (1,)), ((), ())),
                        preferred_element_type=jnp.float32)
    y = (y + b_ref[...]) * dx_ref[...]
    o_ref[0] = y[:, :D2]
    o_ref[1] = y[:, D2:]


def _fin_k(acc_ref, dx_ref, o_ref):
    a = jnp.concatenate([acc_ref[0], acc_ref[1]], axis=1)
    o_ref[...] = a * dx_ref[...]


_BLK = 1000
_G = NN // _BLK

_row_spec = pl.BlockSpec((_BLK, DD), lambda i: (i, 0))
_half_spec = pl.BlockSpec((NC, _BLK, D2), lambda i: (0, i, 0))
_full_spec = pl.BlockSpec((DD, DD), lambda i: (0, 0))
_b_spec = pl.BlockSpec((1, DD), lambda i: (0, 0))

_mm1 = pl.pallas_call(
    _mm1_k, grid=(_G,),
    in_specs=[_row_spec, _full_spec, _b_spec, _row_spec],
    out_specs=_half_spec,
    out_shape=jax.ShapeDtypeStruct((NC, NN, D2), jnp.float32),
)

_mm2 = pl.pallas_call(
    _mm2_k, grid=(_G,),
    in_specs=[_half_spec, _row_spec, _full_spec, _b_spec],
    out_specs=_half_spec,
    out_shape=jax.ShapeDtypeStruct((NC, NN, D2), jnp.float32),
)

_fin = pl.pallas_call(
    _fin_k, grid=(_G,),
    in_specs=[_half_spec, _row_spec],
    out_specs=_row_spec,
    out_shape=jax.ShapeDtypeStruct((NN, DD), jnp.float32),
)


@jax.jit
def kernel(x, edge_index, edge_weight, W1, b1, W2, b2):
    pi = edge_index.astype(jnp.int32).reshape(2, NCHUNK, CH)
    pw = edge_weight.astype(jnp.float32).reshape(NCHUNK, CH)
    z1 = jnp.zeros((ROWS_PER_TILE,), jnp.float32)
    z2 = jnp.zeros((ROWS_PER_TILE, D2), jnp.float32)

    disx = _prep(pi, pw, z1)
    y1s = _mm1(x, W1, b1.reshape(1, DD), disx)
    acc1 = _spmm(y1s, pi, pw, z2)
    y2s = _mm2(acc1, disx, W2, b2.reshape(1, DD))
    acc2 = _spmm(y2s, pi, pw, z2)
    return _fin(acc2, disx)


# speedup vs baseline: 24.1898x; 1.0348x over previous
import jax
import jax.numpy as jnp
from jax import lax
from jax.experimental import pallas as pl
from jax.experimental.pallas import tpu as pltpu
from jax.experimental.pallas import tpu_sc as plsc

NN = 10000
NP = 10240
EE = 320000
DD = 128
D2 = DD // 2
CH = 128
NCHUNK = EE // CH
NC, NS = 2, 16
ROWS_PER_TILE = NP // NS
DIS_PER_TILE = NP // (NC * NS)
NB = 6
NMAIN = (NCHUNK // NS) // NB * NB
NLEFT = NCHUNK - NS * NMAIN

_MESH = plsc.VectorSubcoreMesh(
    core_axis_name="c", subcore_axis_name="s", num_cores=NC, num_subcores=NS)
_SC_PARAMS = pltpu.CompilerParams(needs_layout_passes=False, use_tc_tiling_on_sc=False)


def _rsqrt_newton(d):
    y = lax.bitcast_convert_type(
        jnp.int32(0x5F3759DF)
        - lax.shift_right_logical(lax.bitcast_convert_type(d, jnp.int32), 1),
        jnp.float32)
    for _ in range(4):
        y = y * (1.5 - 0.5 * d * y * y)
    return y


def _prep_body(pi_hbm, pw_hbm, z1_hbm, dis_hbm, *refs):
    pi = refs[0:NB]
    pw = refs[NB:2 * NB]
    semL = refs[2 * NB:3 * NB]
    semS = refs[3 * NB:4 * NB]
    deg_v, deg_sp = refs[4 * NB:4 * NB + 2]
    c = lax.axis_index("c")
    s = lax.axis_index("s")
    pltpu.sync_copy(z1_hbm, deg_sp.at[pl.ds(s * ROWS_PER_TILE, ROWS_PER_TILE)])
    plsc.subcore_barrier()

    def issue_load(i, b):
        g = i * NS + s
        pltpu.async_copy(pi_hbm.at[0].at[g], pi[b].at[0], semL[b])
        pltpu.async_copy(pw_hbm.at[g], pw[b], semL[b])

    def wait_load(b):
        pltpu.make_async_copy(pi_hbm.at[0].at[0], pi[b].at[0], semL[b]).wait()
        pltpu.make_async_copy(pw_hbm.at[0], pw[b], semL[b]).wait()

    def wait_scat(b):
        pltpu.make_async_copy(pw_hbm.at[0], pw[b], semS[b]).wait()

    issue_load(0, 0)
    issue_load(1, 1)

    def outer(t, carry):
        for b in range(NB):
            i = t * NB + b
            b2 = (b + 2) % NB

            @pl.when(i < NMAIN - 2)
            def _():
                @pl.when(i >= NB - 2)
                def _():
                    wait_scat(b2)
                issue_load(i + 2, b2)

            wait_load(b)
            pltpu.async_copy(pw[b], deg_sp.at[pi[b].at[0]], semS[b], add=True)
        return carry

    lax.fori_loop(0, NMAIN // NB, outer, 0)
    for b in range(NB):
        wait_scat(b)

    @pl.when(s < NLEFT)
    def _():
        g = NS * NMAIN + s
        pltpu.sync_copy(pi_hbm.at[0].at[g], pi[0].at[0])
        pltpu.sync_copy(pw_hbm.at[g], pw[0])
        pltpu.sync_copy(pw[0], deg_sp.at[pi[0].at[0]], add=True)

    plsc.subcore_barrier()

    base = (c * NS + s) * DIS_PER_TILE
    pltpu.sync_copy(deg_sp.at[pl.ds(base, DIS_PER_TILE)], deg_v)

    @plsc.parallel_loop(0, DIS_PER_TILE // 16, step=1, unroll=4)
    def newton(t):
        sl = pl.ds(t * 16, 16)
        deg_v[sl] = _rsqrt_newton(deg_v[sl])

    pltpu.sync_copy(deg_v, dis_hbm.at[pl.ds(base, DIS_PER_TILE)])


def _spmm_body(tab_hbm, pi_hbm, pw_hbm, z2_hbm, dis_hbm, acc_hbm, *refs):
    pi = refs[0:NB]
    pw = refs[NB:2 * NB]
    gath = refs[2 * NB:3 * NB]
    semL = refs[3 * NB:4 * NB]
    semG = refs[4 * NB:5 * NB]
    semS = refs[5 * NB:6 * NB]
    acc_sp = refs[6 * NB]
    dis_v = refs[6 * NB + 1]
    wb_v = refs[6 * NB + 2]
    c = lax.axis_index("c")
    s = lax.axis_index("s")
    tabc = tab_hbm.at[c]
    pltpu.sync_copy(z2_hbm, acc_sp.at[pl.ds(s * ROWS_PER_TILE, ROWS_PER_TILE)])
    pltpu.sync_copy(dis_hbm, dis_v)
    plsc.subcore_barrier()

    def issue_load(i, b):
        g = i * NS + s
        pltpu.async_copy(pi_hbm.at[0].at[g], pi[b].at[0], semL[b])
        pltpu.async_copy(pi_hbm.at[1].at[g], pi[b].at[1], semL[b])
        pltpu.async_copy(pw_hbm.at[g], pw[b], semL[b])

    def wait_load(b):
        pltpu.make_async_copy(pi_hbm.at[0].at[0], pi[b].at[0], semL[b]).wait()
        pltpu.make_async_copy(pi_hbm.at[0].at[0], pi[b].at[1], semL[b]).wait()
        pltpu.make_async_copy(pw_hbm.at[0], pw[b], semL[b]).wait()

    def wait_g(sem, b):
        pltpu.make_async_copy(tabc.at[pl.ds(0, CH)], gath[b], sem).wait()

    def scale(b):
        @plsc.parallel_loop(0, CH // 16, step=1, unroll=2)
        def wrow(g):
            sl = pl.ds(g * 16, 16)
            r16 = pi[b][0, sl]
            pw[b][sl] = pw[b][sl] * plsc.load_gather(dis_v, [r16])

        @plsc.parallel_loop(0, CH, step=1, unroll=4)
        def body(jj):
            wj = plsc.load_gather(pw[b], [jnp.full((16,), jj, jnp.int32)])
            for k in range(D2 // 16):
                sl = pl.ds(k * 16, 16)
                gath[b][jj, sl] = gath[b][jj, sl] * wj

    issue_load(0, 0)
    issue_load(1, 1)
    issue_load(2, 2)
    wait_load(0)
    pltpu.async_copy(tabc.at[pi[0].at[0]], gath[0], semG[0])
    wait_load(1)
    pltpu.async_copy(tabc.at[pi[1].at[0]], gath[1], semG[1])

    def outer(t, carry):
        for b in range(NB):
            i = t * NB + b
            b2, b3 = (b + 2) % NB, (b + 3) % NB

            @pl.when(i < NMAIN - 3)
            def _():
                @pl.when(i >= NB - 3)
                def _():
                    wait_g(semS[b3], b3)
                issue_load(i + 3, b3)

            @pl.when(i < NMAIN - 2)
            def _():
                wait_load(b2)
                pltpu.async_copy(tabc.at[pi[b2].at[0]], gath[b2], semG[b2])

            wait_g(semG[b], b)
            scale(b)
            pltpu.async_copy(gath[b], acc_sp.at[pi[b].at[1]], semS[b],
                             add=True)
        return carry

    lax.fori_loop(0, NMAIN // NB, outer, 0)
    for i in range(NMAIN - NB, NMAIN):
        wait_g(semS[i % NB], i % NB)

    @pl.when(s < NLEFT)
    def _():
        g = NS * NMAIN + s
        pltpu.sync_copy(pi_hbm.at[0].at[g], pi[0].at[0])
        pltpu.sync_copy(pi_hbm.at[1].at[g], pi[0].at[1])
        pltpu.sync_copy(pw_hbm.at[g], pw[0])
        pltpu.async_copy(tabc.at[pi[0].at[0]], gath[0], semG[0]).wait()
        scale(0)
        pltpu.sync_copy(gath[0], acc_sp.at[pi[0].at[1]], add=True)

    plsc.subcore_barrier()

    for half in range(2):
        base_r = s * ROWS_PER_TILE + half * (ROWS_PER_TILE // 2)
        slr = pl.ds(base_r, ROWS_PER_TILE // 2)
        pltpu.sync_copy(acc_sp.at[slr], wb_v)

        @plsc.parallel_loop(0, ROWS_PER_TILE // 2, step=1, unroll=4)
        def wb(n):
            dv = plsc.load_gather(dis_v, [jnp.full((16,), base_r + n,
                                                   jnp.int32)])
            for k in range(D2 // 16):
                sl = pl.ds(k * 16, 16)
                wb_v[n, sl] = wb_v[n, sl] * dv

        pltpu.sync_copy(wb_v, acc_hbm.at[c].at[slr])


_prep = pl.kernel(
    _prep_body,
    out_type=jax.ShapeDtypeStruct((NP,), jnp.float32),
    mesh=_MESH,
    compiler_params=_SC_PARAMS,
    scratch_types=(
        [pltpu.VMEM((2, CH), jnp.int32) for _ in range(NB)]
        + [pltpu.VMEM((CH,), jnp.float32) for _ in range(NB)]
        + [pltpu.SemaphoreType.DMA for _ in range(2 * NB)]
        + [pltpu.VMEM((DIS_PER_TILE,), jnp.float32),
           pltpu.VMEM_SHARED((NP,), jnp.float32)]
    ),
)

_spmm = pl.kernel(
    _spmm_body,
    out_type=jax.ShapeDtypeStruct((NC, NP, D2), jnp.float32),
    mesh=_MESH,
    compiler_params=_SC_PARAMS,
    scratch_types=(
        [pltpu.VMEM((2, CH), jnp.int32) for _ in range(NB)]
        + [pltpu.VMEM((CH,), jnp.float32) for _ in range(NB)]
        + [pltpu.VMEM((CH, D2), jnp.float32) for _ in range(NB)]
        + [pltpu.SemaphoreType.DMA for _ in range(3 * NB)]
        + [pltpu.VMEM_SHARED((NP, D2), jnp.float32),
           pltpu.VMEM((NP,), jnp.float32),
           pltpu.VMEM((ROWS_PER_TILE // 2, D2), jnp.float32)]
    ),
)


def _mm1_k(x_ref, w_ref, b_ref, o_ref):
    y = lax.dot_general(x_ref[...], w_ref[...], (((1,), (1,)), ((), ())),
                        preferred_element_type=jnp.float32)
    y = y + b_ref[...]
    o_ref[0] = y[:, :D2]
    o_ref[1] = y[:, D2:]


def _mm2_k(acc_ref, w_ref, b_ref, o_ref):
    a = jnp.concatenate([acc_ref[0], acc_ref[1]], axis=1)
    h = jnp.maximum(a, 0.0)
    y = lax.dot_general(h, w_ref[...], (((1,), (1,)), ((), ())),
                        preferred_element_type=jnp.float32)
    y = y + b_ref[...]
    o_ref[0] = y[:, :D2]
    o_ref[1] = y[:, D2:]


def _fin_k(acc_ref, o_ref):
    o_ref[...] = jnp.concatenate([acc_ref[0], acc_ref[1]], axis=1)


_BLK = 1000
_G = NN // _BLK

_row_spec = pl.BlockSpec((_BLK, DD), lambda i: (i, 0))
_half_spec = pl.BlockSpec((NC, _BLK, D2), lambda i: (0, i, 0))
_full_spec = pl.BlockSpec((DD, DD), lambda i: (0, 0))
_b_spec = pl.BlockSpec((1, DD), lambda i: (0, 0))

_mm1 = pl.pallas_call(
    _mm1_k, grid=(_G,),
    in_specs=[_row_spec, _full_spec, _b_spec],
    out_specs=_half_spec,
    out_shape=jax.ShapeDtypeStruct((NC, NN, D2), jnp.float32),
)

_mm2 = pl.pallas_call(
    _mm2_k, grid=(_G,),
    in_specs=[_half_spec, _full_spec, _b_spec],
    out_specs=_half_spec,
    out_shape=jax.ShapeDtypeStruct((NC, NN, D2), jnp.float32),
)

_fin = pl.pallas_call(
    _fin_k, grid=(_G,),
    in_specs=[_half_spec],
    out_specs=_row_spec,
    out_shape=jax.ShapeDtypeStruct((NN, DD), jnp.float32),
)


@jax.jit
def kernel(x, edge_index, edge_weight, W1, b1, W2, b2):
    pi = edge_index.astype(jnp.int32).reshape(2, NCHUNK, CH)
    pw = edge_weight.astype(jnp.float32).reshape(NCHUNK, CH)
    z1 = jnp.zeros((ROWS_PER_TILE,), jnp.float32)
    z2 = jnp.zeros((ROWS_PER_TILE, D2), jnp.float32)

    dis = _prep(pi, pw, z1)
    y1 = _mm1(x, W1, b1.reshape(1, DD))
    acc1 = _spmm(y1, pi, pw, z2, dis)
    y2 = _mm2(acc1, W2, b2.reshape(1, DD))
    acc2 = _spmm(y2, pi, pw, z2, dis)
    return _fin(acc2)


# speedup vs baseline: 26.7390x; 1.1054x over previous
import functools

import jax
import jax.numpy as jnp
from jax import lax
from jax.experimental import pallas as pl
from jax.experimental.pallas import tpu as pltpu
from jax.experimental.pallas import tpu_sc as plsc

NN = 10000
NP = 10240
EE = 320000
DD = 128
D2 = DD // 2
CH = 128
NCHUNK = EE // CH
NC, NS = 2, 16
ROWS_PER_TILE = NP // NS
DIS_PER_TILE = NP // (NC * NS)
NB = 6
NMAIN = (NCHUNK // NS) // NB * NB
NLEFT = NCHUNK - NS * NMAIN

_MESH = plsc.VectorSubcoreMesh(
    core_axis_name="c", subcore_axis_name="s", num_cores=NC, num_subcores=NS)
_SC_PARAMS = pltpu.CompilerParams(needs_layout_passes=False, use_tc_tiling_on_sc=False)


def _rsqrt_newton(d):
    y = lax.bitcast_convert_type(
        jnp.int32(0x5F3759DF)
        - lax.shift_right_logical(lax.bitcast_convert_type(d, jnp.int32), 1),
        jnp.float32)
    for _ in range(4):
        y = y * (1.5 - 0.5 * d * y * y)
    return y


def _prep_body(pi_hbm, pw_hbm, z1_hbm, dis_hbm, *refs):
    NB3 = 3
    NG = (NCHUNK // 4) // NS
    pi4 = refs[0:NB3]
    pw4 = refs[NB3:2 * NB3]
    semL = refs[2 * NB3:3 * NB3]
    semS = refs[3 * NB3:4 * NB3]
    deg_v, deg_sp = refs[4 * NB3:4 * NB3 + 2]
    c = lax.axis_index("c")
    s = lax.axis_index("s")
    pltpu.sync_copy(z1_hbm, deg_sp.at[pl.ds(s * ROWS_PER_TILE, ROWS_PER_TILE)])
    plsc.subcore_barrier()

    def issue_load(j, b):
        gg = (j * NS + s) * 4
        pltpu.async_copy(pi_hbm.at[0].at[pl.ds(gg, 4)], pi4[b], semL[b])
        pltpu.async_copy(pw_hbm.at[pl.ds(gg, 4)], pw4[b], semL[b])

    def wait_load(b):
        pltpu.make_async_copy(pi_hbm.at[0].at[pl.ds(0, 4)], pi4[b],
                              semL[b]).wait()
        pltpu.make_async_copy(pw_hbm.at[pl.ds(0, 4)], pw4[b], semL[b]).wait()

    def wait_scat(b):
        for k in range(4):
            pltpu.make_async_copy(pw_hbm.at[0], pw4[b].at[k], semS[b]).wait()

    issue_load(0, 0)

    def outer(t, carry):
        for b in range(NB3):
            j = t * NB3 + b
            b1 = (b + 1) % NB3

            @pl.when(j < NG - 1)
            def _():
                @pl.when(j >= NB3 - 1)
                def _():
                    wait_scat(b1)
                issue_load(j + 1, b1)

            wait_load(b)
            for k in range(4):
                pltpu.async_copy(pw4[b].at[k], deg_sp.at[pi4[b].at[k]],
                                 semS[b], add=True)
        return carry

    lax.fori_loop(0, NG // NB3, outer, 0)
    wait_scat((NG - 2) % NB3)
    wait_scat((NG - 1) % NB3)

    @pl.when(s == 0)
    def _():
        gg = NG * NS * 4
        pltpu.sync_copy(pi_hbm.at[0].at[pl.ds(gg, 4)], pi4[0])
        pltpu.sync_copy(pw_hbm.at[pl.ds(gg, 4)], pw4[0])
        for k in range(4):
            pltpu.sync_copy(pw4[0].at[k], deg_sp.at[pi4[0].at[k]], add=True)

    plsc.subcore_barrier()

    base = (c * NS + s) * DIS_PER_TILE
    pltpu.sync_copy(deg_sp.at[pl.ds(base, DIS_PER_TILE)], deg_v)

    @plsc.parallel_loop(0, DIS_PER_TILE // 16, step=1, unroll=4)
    def newton(t):
        sl = pl.ds(t * 16, 16)
        deg_v[sl] = _rsqrt_newton(deg_v[sl])

    pltpu.sync_copy(deg_v, dis_hbm.at[pl.ds(base, DIS_PER_TILE)])


def _spmm_body(direct_out, tab_hbm, pi_hbm, pw_hbm, z2_hbm, dis_hbm, acc_hbm, *refs):
    pi = refs[0:NB]
    pw = refs[NB:2 * NB]
    gath = refs[2 * NB:3 * NB]
    semL = refs[3 * NB:4 * NB]
    semG = refs[4 * NB:5 * NB]
    semS = refs[5 * NB:6 * NB]
    acc_sp = refs[6 * NB]
    dis_v = refs[6 * NB + 1]
    wb_v = refs[6 * NB + 2]
    c = lax.axis_index("c")
    s = lax.axis_index("s")
    tabc = tab_hbm.at[c]
    pltpu.sync_copy(z2_hbm, acc_sp.at[pl.ds(s * ROWS_PER_TILE, ROWS_PER_TILE)])
    pltpu.sync_copy(dis_hbm, dis_v)
    plsc.subcore_barrier()

    def issue_load(i, b):
        g = i * NS + s
        pltpu.async_copy(pi_hbm.at[0].at[g], pi[b].at[0], semL[b])
        pltpu.async_copy(pi_hbm.at[1].at[g], pi[b].at[1], semL[b])
        pltpu.async_copy(pw_hbm.at[g], pw[b], semL[b])

    def wait_load(b):
        pltpu.make_async_copy(pi_hbm.at[0].at[0], pi[b].at[0], semL[b]).wait()
        pltpu.make_async_copy(pi_hbm.at[0].at[0], pi[b].at[1], semL[b]).wait()
        pltpu.make_async_copy(pw_hbm.at[0], pw[b], semL[b]).wait()

    def wait_g(sem, b):
        pltpu.make_async_copy(tabc.at[pl.ds(0, CH)], gath[b], sem).wait()

    def scale(b):
        @plsc.parallel_loop(0, CH // 16, step=1, unroll=2)
        def wrow(g):
            sl = pl.ds(g * 16, 16)
            r16 = pi[b][0, sl]
            pw[b][sl] = pw[b][sl] * plsc.load_gather(dis_v, [r16])

        @plsc.parallel_loop(0, CH, step=1, unroll=4)
        def body(jj):
            wj = plsc.load_gather(pw[b], [jnp.full((16,), jj, jnp.int32)])
            for k in range(D2 // 16):
                sl = pl.ds(k * 16, 16)
                gath[b][jj, sl] = gath[b][jj, sl] * wj

    issue_load(0, 0)
    issue_load(1, 1)
    issue_load(2, 2)
    wait_load(0)
    pltpu.async_copy(tabc.at[pi[0].at[0]], gath[0], semG[0])
    wait_load(1)
    pltpu.async_copy(tabc.at[pi[1].at[0]], gath[1], semG[1])

    def outer(t, carry):
        for b in range(NB):
            i = t * NB + b
            b2, b3 = (b + 2) % NB, (b + 3) % NB

            @pl.when(i < NMAIN - 3)
            def _():
                @pl.when(i >= NB - 3)
                def _():
                    wait_g(semS[b3], b3)
                issue_load(i + 3, b3)

            @pl.when(i < NMAIN - 2)
            def _():
                wait_load(b2)
                pltpu.async_copy(tabc.at[pi[b2].at[0]], gath[b2], semG[b2])

            wait_g(semG[b], b)
            scale(b)
            pltpu.async_copy(gath[b], acc_sp.at[pi[b].at[1]], semS[b],
                             add=True)
        return carry

    lax.fori_loop(0, NMAIN // NB, outer, 0)
    for i in range(NMAIN - NB, NMAIN):
        wait_g(semS[i % NB], i % NB)

    @pl.when(s < NLEFT)
    def _():
        g = NS * NMAIN + s
        pltpu.sync_copy(pi_hbm.at[0].at[g], pi[0].at[0])
        pltpu.sync_copy(pi_hbm.at[1].at[g], pi[0].at[1])
        pltpu.sync_copy(pw_hbm.at[g], pw[0])
        pltpu.async_copy(tabc.at[pi[0].at[0]], gath[0], semG[0]).wait()
        scale(0)
        pltpu.sync_copy(gath[0], acc_sp.at[pi[0].at[1]], add=True)

    plsc.subcore_barrier()

    for half in range(2):
        base_r = s * ROWS_PER_TILE + half * (ROWS_PER_TILE // 2)
        slr = pl.ds(base_r, ROWS_PER_TILE // 2)
        pltpu.sync_copy(acc_sp.at[slr], wb_v)

        @plsc.parallel_loop(0, ROWS_PER_TILE // 2, step=1, unroll=4)
        def wb(n):
            dv = plsc.load_gather(dis_v, [jnp.full((16,), base_r + n,
                                                   jnp.int32)])
            for k in range(D2 // 16):
                sl = pl.ds(k * 16, 16)
                wb_v[n, sl] = wb_v[n, sl] * dv

        if not direct_out:
            pltpu.sync_copy(wb_v, acc_hbm.at[c].at[slr])
        elif half == 0:
            pltpu.sync_copy(
                wb_v, acc_hbm.at[slr, pl.ds(c * D2, D2)])
        else:
            @pl.when(s < NS - 1)
            def _():
                pltpu.sync_copy(
                    wb_v, acc_hbm.at[slr, pl.ds(c * D2, D2)])

            @pl.when(s == NS - 1)
            def _():
                nlast = NN - (NP - ROWS_PER_TILE // 2)
                pltpu.sync_copy(
                    wb_v.at[pl.ds(0, nlast)],
                    acc_hbm.at[pl.ds(NP - ROWS_PER_TILE // 2, nlast),
                               pl.ds(c * D2, D2)])


_prep = pl.kernel(
    _prep_body,
    out_type=jax.ShapeDtypeStruct((NP,), jnp.float32),
    mesh=_MESH,
    compiler_params=_SC_PARAMS,
    scratch_types=(
        [pltpu.VMEM((4, CH), jnp.int32) for _ in range(3)]
        + [pltpu.VMEM((4, CH), jnp.float32) for _ in range(3)]
        + [pltpu.SemaphoreType.DMA for _ in range(6)]
        + [pltpu.VMEM((DIS_PER_TILE,), jnp.float32),
           pltpu.VMEM_SHARED((NP,), jnp.float32)]
    ),
)

_SPMM_SCRATCH = (
        [pltpu.VMEM((2, CH), jnp.int32) for _ in range(NB)]
        + [pltpu.VMEM((CH,), jnp.float32) for _ in range(NB)]
        + [pltpu.VMEM((CH, D2), jnp.float32) for _ in range(NB)]
        + [pltpu.SemaphoreType.DMA for _ in range(3 * NB)]
        + [pltpu.VMEM_SHARED((NP, D2), jnp.float32),
           pltpu.VMEM((NP,), jnp.float32),
           pltpu.VMEM((ROWS_PER_TILE // 2, D2), jnp.float32)]
)

_spmm = pl.kernel(
    functools.partial(_spmm_body, False),
    out_type=jax.ShapeDtypeStruct((NC, NP, D2), jnp.float32),
    mesh=_MESH,
    compiler_params=_SC_PARAMS,
    scratch_types=_SPMM_SCRATCH,
)

_spmm_out = pl.kernel(
    functools.partial(_spmm_body, True),
    out_type=jax.ShapeDtypeStruct((NN, DD), jnp.float32),
    mesh=_MESH,
    compiler_params=_SC_PARAMS,
    scratch_types=_SPMM_SCRATCH,
)


def _mm1_k(x_ref, w_ref, b_ref, o_ref):
    y = lax.dot_general(x_ref[...], w_ref[...], (((1,), (1,)), ((), ())),
                        preferred_element_type=jnp.float32)
    y = y + b_ref[...]
    o_ref[0] = y[:, :D2]
    o_ref[1] = y[:, D2:]


def _mm2_k(acc_ref, w_ref, b_ref, o_ref):
    a = jnp.concatenate([acc_ref[0], acc_ref[1]], axis=1)
    h = jnp.maximum(a, 0.0)
    y = lax.dot_general(h, w_ref[...], (((1,), (1,)), ((), ())),
                        preferred_element_type=jnp.float32)
    y = y + b_ref[...]
    o_ref[0] = y[:, :D2]
    o_ref[1] = y[:, D2:]


_BLK = 1000
_G = NN // _BLK

_row_spec = pl.BlockSpec((_BLK, DD), lambda i: (i, 0))
_half_spec = pl.BlockSpec((NC, _BLK, D2), lambda i: (0, i, 0))
_full_spec = pl.BlockSpec((DD, DD), lambda i: (0, 0))
_b_spec = pl.BlockSpec((1, DD), lambda i: (0, 0))

_mm1 = pl.pallas_call(
    _mm1_k, grid=(_G,),
    in_specs=[_row_spec, _full_spec, _b_spec],
    out_specs=_half_spec,
    out_shape=jax.ShapeDtypeStruct((NC, NN, D2), jnp.float32),
)

_mm2 = pl.pallas_call(
    _mm2_k, grid=(_G,),
    in_specs=[_half_spec, _full_spec, _b_spec],
    out_specs=_half_spec,
    out_shape=jax.ShapeDtypeStruct((NC, NN, D2), jnp.float32),
)


@jax.jit
def kernel(x, edge_index, edge_weight, W1, b1, W2, b2):
    pi = edge_index.astype(jnp.int32).reshape(2, NCHUNK, CH)
    pw = edge_weight.astype(jnp.float32).reshape(NCHUNK, CH)
    z1 = jnp.zeros((ROWS_PER_TILE,), jnp.float32)
    z2 = jnp.zeros((ROWS_PER_TILE, D2), jnp.float32)

    dis = _prep(pi, pw, z1)
    y1 = _mm1(x, W1, b1.reshape(1, DD))
    acc1 = _spmm(y1, pi, pw, z2, dis)
    y2 = _mm2(acc1, W2, b2.reshape(1, DD))
    return _spmm_out(y2, pi, pw, z2, dis)


# speedup vs baseline: 26.7899x; 1.0019x over previous
import functools

import jax
import jax.numpy as jnp
from jax import lax
from jax.experimental import pallas as pl
from jax.experimental.pallas import tpu as pltpu
from jax.experimental.pallas import tpu_sc as plsc

NN = 10000
NP = 10240
EE = 320000
DD = 128
D2 = DD // 2
CH = 128
NCHUNK = EE // CH
NC, NS = 2, 16
ROWS_PER_TILE = NP // NS
DIS_PER_TILE = NP // (NC * NS)
NB = 6
NMAIN = (NCHUNK // NS) // NB * NB
NLEFT = NCHUNK - NS * NMAIN

_MESH = plsc.VectorSubcoreMesh(
    core_axis_name="c", subcore_axis_name="s", num_cores=NC, num_subcores=NS)
_SC_PARAMS = pltpu.CompilerParams(needs_layout_passes=False, use_tc_tiling_on_sc=False)


def _rsqrt_newton(d):
    y = lax.bitcast_convert_type(
        jnp.int32(0x5F3759DF)
        - lax.shift_right_logical(lax.bitcast_convert_type(d, jnp.int32), 1),
        jnp.float32)
    for _ in range(4):
        y = y * (1.5 - 0.5 * d * y * y)
    return y


def _prep_body(pi_hbm, pw_hbm, z1_hbm, dis_hbm, *refs):
    NB3 = 3
    NG = (NCHUNK // 4) // NS
    pi4 = refs[0:NB3]
    pw4 = refs[NB3:2 * NB3]
    semL = refs[2 * NB3:3 * NB3]
    semS = refs[3 * NB3:4 * NB3]
    deg_v, deg_sp = refs[4 * NB3:4 * NB3 + 2]
    c = lax.axis_index("c")
    s = lax.axis_index("s")
    pltpu.sync_copy(z1_hbm, deg_sp.at[pl.ds(s * ROWS_PER_TILE, ROWS_PER_TILE)])
    plsc.subcore_barrier()

    def issue_load(j, b):
        gg = (j * NS + s) * 4
        pltpu.async_copy(pi_hbm.at[0].at[pl.ds(gg, 4)], pi4[b], semL[b])
        pltpu.async_copy(pw_hbm.at[pl.ds(gg, 4)], pw4[b], semL[b])

    def wait_load(b):
        pltpu.make_async_copy(pi_hbm.at[0].at[pl.ds(0, 4)], pi4[b],
                              semL[b]).wait()
        pltpu.make_async_copy(pw_hbm.at[pl.ds(0, 4)], pw4[b], semL[b]).wait()

    def wait_scat(b):
        for k in range(4):
            pltpu.make_async_copy(pw_hbm.at[0], pw4[b].at[k], semS[b]).wait()

    issue_load(0, 0)

    def outer(t, carry):
        for b in range(NB3):
            j = t * NB3 + b
            b1 = (b + 1) % NB3

            @pl.when(j < NG - 1)
            def _():
                @pl.when(j >= NB3 - 1)
                def _():
                    wait_scat(b1)
                issue_load(j + 1, b1)

            wait_load(b)
            for k in range(4):
                pltpu.async_copy(pw4[b].at[k], deg_sp.at[pi4[b].at[k]],
                                 semS[b], add=True)
        return carry

    lax.fori_loop(0, NG // NB3, outer, 0)
    for j in range(NG - 3, NG):
        wait_scat(j % NB3)

    @pl.when(s == 0)
    def _():
        gg = NG * NS * 4
        pltpu.sync_copy(pi_hbm.at[0].at[pl.ds(gg, 4)], pi4[0])
        pltpu.sync_copy(pw_hbm.at[pl.ds(gg, 4)], pw4[0])
        for k in range(4):
            pltpu.sync_copy(pw4[0].at[k], deg_sp.at[pi4[0].at[k]], add=True)

    plsc.subcore_barrier()

    base = (c * NS + s) * DIS_PER_TILE
    pltpu.sync_copy(deg_sp.at[pl.ds(base, DIS_PER_TILE)], deg_v)

    @plsc.parallel_loop(0, DIS_PER_TILE // 16, step=1, unroll=4)
    def newton(t):
        sl = pl.ds(t * 16, 16)
        deg_v[sl] = _rsqrt_newton(deg_v[sl])

    pltpu.sync_copy(deg_v, dis_hbm.at[pl.ds(base, DIS_PER_TILE)])


def _spmm_body(direct_out, tab_hbm, pi_hbm, pw_hbm, z2_hbm, dis_hbm, acc_hbm, *refs):
    pi = refs[0:NB]
    pw = refs[NB:2 * NB]
    gath = refs[2 * NB:3 * NB]
    semL = refs[3 * NB:4 * NB]
    semG = refs[4 * NB:5 * NB]
    semS = refs[5 * NB:6 * NB]
    acc_sp = refs[6 * NB]
    dis_v = refs[6 * NB + 1]
    wb_v = refs[6 * NB + 2]
    c = lax.axis_index("c")
    s = lax.axis_index("s")
    tabc = tab_hbm.at[c]
    pltpu.sync_copy(z2_hbm, acc_sp.at[pl.ds(s * ROWS_PER_TILE, ROWS_PER_TILE)])
    pltpu.sync_copy(dis_hbm, dis_v)
    plsc.subcore_barrier()

    def issue_load(i, b):
        g = i * NS + s
        pltpu.async_copy(pi_hbm.at[0].at[g], pi[b].at[0], semL[b])
        pltpu.async_copy(pi_hbm.at[1].at[g], pi[b].at[1], semL[b])
        pltpu.async_copy(pw_hbm.at[g], pw[b], semL[b])

    def wait_load(b):
        pltpu.make_async_copy(pi_hbm.at[0].at[0], pi[b].at[0], semL[b]).wait()
        pltpu.make_async_copy(pi_hbm.at[0].at[0], pi[b].at[1], semL[b]).wait()
        pltpu.make_async_copy(pw_hbm.at[0], pw[b], semL[b]).wait()

    def wait_g(sem, b):
        pltpu.make_async_copy(tabc.at[pl.ds(0, CH)], gath[b], sem).wait()

    def scale(b):
        @plsc.parallel_loop(0, CH // 16, step=1, unroll=2)
        def wrow(g):
            sl = pl.ds(g * 16, 16)
            r16 = pi[b][0, sl]
            pw[b][sl] = pw[b][sl] * plsc.load_gather(dis_v, [r16])

        @plsc.parallel_loop(0, CH, step=1, unroll=4)
        def body(jj):
            wj = plsc.load_gather(pw[b], [jnp.full((16,), jj, jnp.int32)])
            for k in range(D2 // 16):
                sl = pl.ds(k * 16, 16)
                gath[b][jj, sl] = gath[b][jj, sl] * wj

    issue_load(0, 0)
    issue_load(1, 1)
    issue_load(2, 2)
    wait_load(0)
    pltpu.async_copy(tabc.at[pi[0].at[0]], gath[0], semG[0])
    wait_load(1)
    pltpu.async_copy(tabc.at[pi[1].at[0]], gath[1], semG[1])

    def outer(t, carry):
        for b in range(NB):
            i = t * NB + b
            b2, b3 = (b + 2) % NB, (b + 3) % NB

            @pl.when(i < NMAIN - 3)
            def _():
                @pl.when(i >= NB - 3)
                def _():
                    wait_g(semS[b3], b3)
                issue_load(i + 3, b3)

            @pl.when(i < NMAIN - 2)
            def _():
                wait_load(b2)
                pltpu.async_copy(tabc.at[pi[b2].at[0]], gath[b2], semG[b2])

            wait_g(semG[b], b)
            scale(b)
            pltpu.async_copy(gath[b], acc_sp.at[pi[b].at[1]], semS[b],
                             add=True)
        return carry

    lax.fori_loop(0, NMAIN // NB, outer, 0)
    for i in range(NMAIN - NB, NMAIN):
        wait_g(semS[i % NB], i % NB)

    @pl.when(s < NLEFT)
    def _():
        g = NS * NMAIN + s
        pltpu.sync_copy(pi_hbm.at[0].at[g], pi[0].at[0])
        pltpu.sync_copy(pi_hbm.at[1].at[g], pi[0].at[1])
        pltpu.sync_copy(pw_hbm.at[g], pw[0])
        pltpu.async_copy(tabc.at[pi[0].at[0]], gath[0], semG[0]).wait()
        scale(0)
        pltpu.sync_copy(gath[0], acc_sp.at[pi[0].at[1]], add=True)

    plsc.subcore_barrier()

    for half in range(2):
        base_r = s * ROWS_PER_TILE + half * (ROWS_PER_TILE // 2)
        slr = pl.ds(base_r, ROWS_PER_TILE // 2)
        pltpu.sync_copy(acc_sp.at[slr], wb_v)

        @plsc.parallel_loop(0, ROWS_PER_TILE // 2, step=1, unroll=4)
        def wb(n):
            dv = plsc.load_gather(dis_v, [jnp.full((16,), base_r + n,
                                                   jnp.int32)])
            for k in range(D2 // 16):
                sl = pl.ds(k * 16, 16)
                wb_v[n, sl] = wb_v[n, sl] * dv

        if not direct_out:
            pltpu.sync_copy(wb_v, acc_hbm.at[c].at[slr])
        elif half == 0:
            pltpu.sync_copy(
                wb_v, acc_hbm.at[slr, pl.ds(c * D2, D2)])
        else:
            @pl.when(s < NS - 1)
            def _():
                pltpu.sync_copy(
                    wb_v, acc_hbm.at[slr, pl.ds(c * D2, D2)])

            @pl.when(s == NS - 1)
            def _():
                nlast = NN - (NP - ROWS_PER_TILE // 2)
                pltpu.sync_copy(
                    wb_v.at[pl.ds(0, nlast)],
                    acc_hbm.at[pl.ds(NP - ROWS_PER_TILE // 2, nlast),
                               pl.ds(c * D2, D2)])


_prep = pl.kernel(
    _prep_body,
    out_type=jax.ShapeDtypeStruct((NP,), jnp.float32),
    mesh=_MESH,
    compiler_params=_SC_PARAMS,
    scratch_types=(
        [pltpu.VMEM((4, CH), jnp.int32) for _ in range(3)]
        + [pltpu.VMEM((4, CH), jnp.float32) for _ in range(3)]
        + [pltpu.SemaphoreType.DMA for _ in range(6)]
        + [pltpu.VMEM((DIS_PER_TILE,), jnp.float32),
           pltpu.VMEM_SHARED((NP,), jnp.float32)]
    ),
)

_SPMM_SCRATCH = (
        [pltpu.VMEM((2, CH), jnp.int32) for _ in range(NB)]
        + [pltpu.VMEM((CH,), jnp.float32) for _ in range(NB)]
        + [pltpu.VMEM((CH, D2), jnp.float32) for _ in range(NB)]
        + [pltpu.SemaphoreType.DMA for _ in range(3 * NB)]
        + [pltpu.VMEM_SHARED((NP, D2), jnp.float32),
           pltpu.VMEM((NP,), jnp.float32),
           pltpu.VMEM((ROWS_PER_TILE // 2, D2), jnp.float32)]
)

_spmm = pl.kernel(
    functools.partial(_spmm_body, False),
    out_type=jax.ShapeDtypeStruct((NC, NP, D2), jnp.float32),
    mesh=_MESH,
    compiler_params=_SC_PARAMS,
    scratch_types=_SPMM_SCRATCH,
)

_spmm_out = pl.kernel(
    functools.partial(_spmm_body, True),
    out_type=jax.ShapeDtypeStruct((NN, DD), jnp.float32),
    mesh=_MESH,
    compiler_params=_SC_PARAMS,
    scratch_types=_SPMM_SCRATCH,
)


def _mm1_k(x_ref, w_ref, b_ref, o_ref):
    y = lax.dot_general(x_ref[...], w_ref[...], (((1,), (1,)), ((), ())),
                        preferred_element_type=jnp.float32)
    y = y + b_ref[...]
    o_ref[0] = y[:, :D2]
    o_ref[1] = y[:, D2:]


def _mm2_k(acc_ref, w_ref, b_ref, o_ref):
    a = jnp.concatenate([acc_ref[0], acc_ref[1]], axis=1)
    h = jnp.maximum(a, 0.0)
    y = lax.dot_general(h, w_ref[...], (((1,), (1,)), ((), ())),
                        preferred_element_type=jnp.float32)
    y = y + b_ref[...]
    o_ref[0] = y[:, :D2]
    o_ref[1] = y[:, D2:]


_BLK = 1000
_G = NN // _BLK

_row_spec = pl.BlockSpec((_BLK, DD), lambda i: (i, 0))
_half_spec = pl.BlockSpec((NC, _BLK, D2), lambda i: (0, i, 0))
_full_spec = pl.BlockSpec((DD, DD), lambda i: (0, 0))
_b_spec = pl.BlockSpec((1, DD), lambda i: (0, 0))

_mm1 = pl.pallas_call(
    _mm1_k, grid=(_G,),
    in_specs=[_row_spec, _full_spec, _b_spec],
    out_specs=_half_spec,
    out_shape=jax.ShapeDtypeStruct((NC, NN, D2), jnp.float32),
)

_mm2 = pl.pallas_call(
    _mm2_k, grid=(_G,),
    in_specs=[_half_spec, _full_spec, _b_spec],
    out_specs=_half_spec,
    out_shape=jax.ShapeDtypeStruct((NC, NN, D2), jnp.float32),
)


@jax.jit
def kernel(x, edge_index, edge_weight, W1, b1, W2, b2):
    pi = edge_index.astype(jnp.int32).reshape(2, NCHUNK, CH)
    pw = edge_weight.astype(jnp.float32).reshape(NCHUNK, CH)
    z1 = jnp.zeros((ROWS_PER_TILE,), jnp.float32)
    z2 = jnp.zeros((ROWS_PER_TILE, D2), jnp.float32)

    dis = _prep(pi, pw, z1)
    y1 = _mm1(x, W1, b1.reshape(1, DD))
    acc1 = _spmm(y1, pi, pw, z2, dis)
    y2 = _mm2(acc1, W2, b2.reshape(1, DD))
    return _spmm_out(y2, pi, pw, z2, dis)
